# baseline (speedup 1.0000x reference)
import functools

import jax
import jax.numpy as jnp
from jax import lax
from jax.experimental import pallas as pl
from jax.experimental.pallas import tpu as pltpu
from jax.experimental.pallas import tpu_sc as plsc

F32 = jnp.float32
BF16 = jnp.bfloat16
NEGL = -1e30
H = 128
H4 = 512


def _mm(a, b):
    return jnp.dot(a.astype(BF16), b.astype(BF16), preferred_element_type=F32)


def _tableproj_body(wv_ref, wt_ref, out_ref):
    out_ref[...] = _mm(wv_ref[...], wt_ref[...])


def _project_table(wv, wprojT):
    Vv, Dd = wv.shape
    Hh = wprojT.shape[1]
    blk = 2000
    return pl.pallas_call(
        _tableproj_body,
        grid=(Vv // blk,),
        in_specs=[pl.BlockSpec((blk, Dd), lambda i: (i, 0)),
                  pl.BlockSpec((Dd, Hh), lambda i: (0, 0))],
        out_specs=pl.BlockSpec((blk, Hh), lambda i: (i, 0)),
        out_shape=jax.ShapeDtypeStruct((Vv, Hh), F32),
    )(wv, wprojT)


def _sc_gather(table, idx):
    info = plsc.get_sparse_core_info()
    nc, ns = info.num_cores, info.num_subcores
    nw = nc * ns
    n = idx.shape[0]
    bpw = n // nw
    Dd = table.shape[1]
    dt = table.dtype
    row_b = Dd * table.dtype.itemsize
    buf_rows = min(bpw, (400 * 1024 // row_b) // 8 * 8)
    chunks = []
    off = 0
    while off < bpw:
        sz = min(buf_rows, bpw - off)
        chunks.append((off, sz))
        off += sz
    mesh = plsc.VectorSubcoreMesh(core_axis_name="c", subcore_axis_name="s")

    @functools.partial(
        pl.kernel, mesh=mesh,
        out_type=jax.ShapeDtypeStruct((n, Dd), dt),
        scratch_types=[pltpu.VMEM((bpw,), jnp.int32),
                       pltpu.VMEM((buf_rows, Dd), dt),
                       pltpu.SemaphoreType.DMA],
    )
    def gk(table_hbm, idx_hbm, out_hbm, idx_v, rows_v, sem):
        wid = lax.axis_index("s") * nc + lax.axis_index("c")
        base = wid * bpw
        pltpu.sync_copy(idx_hbm.at[pl.ds(base, bpw)], idx_v)
        for off, sz in chunks:
            pltpu.async_copy(table_hbm.at[idx_v.at[pl.ds(off, sz)]],
                             rows_v.at[pl.ds(0, sz)], sem).wait()
            pltpu.sync_copy(rows_v.at[pl.ds(0, sz)],
                            out_hbm.at[pl.ds(base + off, sz)])

    return gk(table, idx)


def _embed_body(e_ref, bert_ref, wg1, bg1, wt1, bt1, wg2, bg2, wt2, bt2,
                wb, bb, out_ref):
    x = e_ref[...]
    for wg, bg, wt, bt in ((wg1, bg1, wt1, bt1), (wg2, bg2, wt2, bt2)):
        g = jax.nn.sigmoid(_mm(x, wg[...]) + bg[...])
        t = jnp.maximum(_mm(x, wt[...]) + bt[...], 0.0)
        x = g * t + (1.0 - g) * x
    bh = jnp.maximum(_mm(bert_ref[...], wb[...]) + bb[...], 0.0)
    out_ref[...] = (x * (1.0 + bh)).astype(BF16)


def _embed_hw(e_all, bert_all, p):
    n, Hh = e_all.shape
    Db = bert_all.shape[1]
    blk = 1440
    w = lambda k: p[k].T
    b = lambda k: p[k].reshape(1, -1)
    args = (w('Wg1'), b('bg1'), w('Wt1'), b('bt1'),
            w('Wg2'), b('bg2'), w('Wt2'), b('bt2'),
            w('Wbert'), b('bbert'))
    return pl.pallas_call(
        _embed_body,
        grid=(n // blk,),
        in_specs=[pl.BlockSpec((blk, Hh), lambda i: (i, 0)),
                  pl.BlockSpec((blk, Db), lambda i: (i, 0))]
                 + [pl.BlockSpec(a.shape, lambda i: (0, 0)) for a in args],
        out_specs=pl.BlockSpec((blk, Hh), lambda i: (i, 0)),
        out_shape=jax.ShapeDtypeStruct((n, Hh), BF16),
    )(e_all, bert_all, *args)


def _bilstm_body(TB, Bb, dins, *refs):
    np_ = len(dins)
    xf_refs = refs[0:np_]
    xb_refs = refs[np_:2 * np_]
    mf_ref, mb_ref = refs[2 * np_:2 * np_ + 2]
    wf_refs = refs[2 * np_ + 2:3 * np_ + 2]
    wb_refs = refs[3 * np_ + 2:4 * np_ + 2]
    bf_ref, bb_ref, wc_ref = refs[4 * np_ + 2:4 * np_ + 5]
    outf_ref, outb_ref, xpf_s, xpb_s, hc_s, wcb_s = refs[4 * np_ + 5:]
    j = pl.program_id(0)

    @pl.when(j == 0)
    def _():
        hc_s[...] = jnp.zeros_like(hc_s)

    wcb_s[...] = wc_ref[...].astype(BF16)

    xpf = bf_ref[...]
    xpb = bb_ref[...]
    for xr, wr, d in zip(xf_refs, wf_refs, dins):
        xpf = xpf + _mm(xr[...].reshape(TB * Bb, d), wr[...])
    for xr, wr, d in zip(xb_refs, wb_refs, dins):
        xpb = xpb + _mm(xr[...].reshape(TB * Bb, d), wr[...])
    xpf_s[...] = xpf.reshape(TB, Bb, H4)
    xpb_s[...] = xpb.reshape(TB, Bb, H4)

    def step(k, _):
        kk = TB - 1 - k
        h = hc_s[0]
        c = hc_s[1]
        z64 = jnp.dot(h.astype(BF16), wcb_s[...], preferred_element_type=F32)
        zf = z64[0:Bb, 0:H4] + xpf_s[k]
        zb = z64[Bb:2 * Bb, H4:2 * H4] + xpb_s[kk]
        z = jnp.concatenate([zf, zb], axis=0)
        i_ = jax.nn.sigmoid(z[:, 0:H])
        f_ = jax.nn.sigmoid(z[:, H:2 * H])
        g_ = jnp.tanh(z[:, 2 * H:3 * H])
        o_ = jax.nn.sigmoid(z[:, 3 * H:4 * H])
        c_new = f_ * c + i_ * g_
        h_new = o_ * jnp.tanh(c_new)
        hc_s[0] = h_new
        hc_s[1] = c_new
        outf_ref[k] = h_new[0:Bb].astype(BF16)
        outb_ref[kk] = h_new[Bb:2 * Bb].astype(BF16)
        return 0

    lax.fori_loop(0, TB, step, 0, unroll=16)
    outf_ref[...] = outf_ref[...] * mf_ref[...][:, :, None]
    outb_ref[...] = outb_ref[...] * mb_ref[...][:, :, None]


def _bilstm(x_parts, mask_t, p, TB):
    T, Bb = x_parts[0].shape[:2]
    dins = tuple(x.shape[2] for x in x_parts)
    G = T // TB
    wfT = p['Wih_f'].T
    wbT = p['Wih_b'].T
    offs = [0]
    for d in dins:
        offs.append(offs[-1] + d)
    wf_parts = [wfT[offs[i]:offs[i + 1]] for i in range(len(dins))]
    wb_parts = [wbT[offs[i]:offs[i + 1]] for i in range(len(dins))]
    bf = p['b_f'].reshape(1, -1)
    bb = p['b_b'].reshape(1, -1)
    wc = jnp.concatenate([p['Whh_f'].T, p['Whh_b'].T], axis=1)
    body = functools.partial(_bilstm_body, TB, Bb, dins)
    xspec_f = [pl.BlockSpec((TB, Bb, d), lambda j: (j, 0, 0)) for d in dins]
    xspec_b = [pl.BlockSpec((TB, Bb, d), lambda j, G=G: (G - 1 - j, 0, 0))
               for d in dins]
    wspec = [pl.BlockSpec((d, H4), lambda j: (0, 0)) for d in dins]
    outf, outb = pl.pallas_call(
        body,
        grid=(G,),
        in_specs=xspec_f + xspec_b + [
            pl.BlockSpec((TB, Bb), lambda j: (j, 0)),
            pl.BlockSpec((TB, Bb), lambda j, G=G: (G - 1 - j, 0)),
        ] + wspec + wspec + [
            pl.BlockSpec((1, H4), lambda j: (0, 0)),
            pl.BlockSpec((1, H4), lambda j: (0, 0)),
            pl.BlockSpec((H, 2 * H4), lambda j: (0, 0)),
        ],
        out_specs=[
            pl.BlockSpec((TB, Bb, H), lambda j: (j, 0, 0)),
            pl.BlockSpec((TB, Bb, H), lambda j, G=G: (G - 1 - j, 0, 0)),
        ],
        out_shape=[jax.ShapeDtypeStruct((T, Bb, H), BF16),
                   jax.ShapeDtypeStruct((T, Bb, H), BF16)],
        scratch_shapes=[pltpu.VMEM((TB, Bb, H4), F32),
                        pltpu.VMEM((TB, Bb, H4), F32),
                        pltpu.VMEM((2, 2 * Bb, H), F32),
                        pltpu.VMEM((H, 2 * H4), BF16)],
    )(*x_parts, *x_parts, *(mask_t, mask_t), *wf_parts, *wb_parts, bf, bb, wc)
    return outf, outb


def _att_body(GB, Tc, Tq, cf_ref, cb_ref, qf_ref, qb_ref, cm_ref, qm_ref,
              wq_ref, bq_ref, out_ref):
    cv = jnp.concatenate([cf_ref[...], cb_ref[...]], axis=2)
    c = jnp.transpose(cv, (1, 0, 2))
    q = jnp.transpose(jnp.concatenate([qf_ref[...], qb_ref[...]], axis=2),
                      (1, 0, 2))
    cm = cm_ref[0]
    qm = qm_ref[0]
    D2 = c.shape[2]
    qp = jnp.tanh(_mm(q.reshape(GB * Tq, D2), wq_ref[...]).reshape(GB, Tq, D2)
                  + bq_ref[...])
    Lg = lax.dot_general(c, qp.astype(BF16),
                         (((2,), (2,)), ((0,), (0,))),
                         preferred_element_type=F32)
    La = jnp.where(qm[:, None, :] > 0, Lg, NEGL)
    A = jax.nn.softmax(La, axis=2)
    Lb = jnp.where(cm[:, :, None] > 0, Lg, NEGL)
    Bm = jax.nn.softmax(Lb, axis=1)
    c2q = lax.dot_general(A.astype(BF16), qp.astype(BF16),
                          (((2,), (1,)), ((0,), (0,))),
                          preferred_element_type=F32)
    q2c = lax.dot_general(Bm.astype(BF16), c,
                          (((1,), (1,)), ((0,), (0,))),
                          preferred_element_type=F32)
    coatt = lax.dot_general(A.astype(BF16), q2c.astype(BF16),
                            (((2,), (1,)), ((0,), (0,))),
                            preferred_element_type=F32)
    c2q_t = jnp.transpose(c2q, (1, 0, 2))
    coatt_t = jnp.transpose(coatt, (1, 0, 2))
    cv32 = cv.astype(F32)
    out_ref[:, :, 0:D2] = cv
    out_ref[:, :, D2:2 * D2] = c2q_t.astype(BF16)
    out_ref[:, :, 2 * D2:3 * D2] = (cv32 * c2q_t).astype(BF16)
    out_ref[:, :, 3 * D2:4 * D2] = (cv32 * coatt_t).astype(BF16)


def _attention(c_parts, q_parts, cm_b, qm_b, p):
    Tc, Bb, Hh = c_parts[0].shape
    Tq = q_parts[0].shape[0]
    D2 = 2 * Hh
    GB = 16
    wq = p['Wq'].T
    bq = p['bq'].reshape(1, 1, -1)
    cm3 = cm_b.reshape(Bb // GB, GB, Tc)
    qm3 = qm_b.reshape(Bb // GB, GB, Tq)
    body = functools.partial(_att_body, GB, Tc, Tq)
    return pl.pallas_call(
        body,
        grid=(Bb // GB,),
        in_specs=[
            pl.BlockSpec((Tc, GB, Hh), lambda i: (0, i, 0)),
            pl.BlockSpec((Tc, GB, Hh), lambda i: (0, i, 0)),
            pl.BlockSpec((Tq, GB, Hh), lambda i: (0, i, 0)),
            pl.BlockSpec((Tq, GB, Hh), lambda i: (0, i, 0)),
            pl.BlockSpec((1, GB, Tc), lambda i: (i, 0, 0)),
            pl.BlockSpec((1, GB, Tq), lambda i: (i, 0, 0)),
            pl.BlockSpec((D2, D2), lambda i: (0, 0)),
            pl.BlockSpec((1, 1, D2), lambda i: (0, 0, 0)),
        ],
        out_specs=pl.BlockSpec((Tc, GB, 4 * D2), lambda i: (0, i, 0)),
        out_shape=jax.ShapeDtypeStruct((Tc, Bb, 4 * D2), BF16),
    )(*c_parts, *q_parts, cm3, qm3, wq, bq)


def _logits_body(att_ref, m2f_ref, m2b_ref, mof_ref, mob_ref,
                 wa1, wm1a, wm1b, wa2, wm2a, wm2b, l1_ref, l2_ref):
    att = att_ref[...]
    l1_ref[...] = (jnp.sum(att * wa1[...], axis=2)
                   + jnp.sum(m2f_ref[...] * wm1a[...], axis=2)
                   + jnp.sum(m2b_ref[...] * wm1b[...], axis=2))
    l2_ref[...] = (jnp.sum(att * wa2[...], axis=2)
                   + jnp.sum(mof_ref[...] * wm2a[...], axis=2)
                   + jnp.sum(mob_ref[...] * wm2b[...], axis=2))


def _logits(att, mod_parts, mod2_parts, p):
    Tc, Bb, D8 = att.shape
    TB = 80
    va = lambda k: p[k].reshape(1, 1, -1)
    vh = lambda k, s: p[k].reshape(-1)[s * H:(s + 1) * H].reshape(1, 1, H)
    hspec = pl.BlockSpec((TB, Bb, H), lambda i: (i, 0, 0))
    wspec1 = pl.BlockSpec((1, 1, D8), lambda i: (0, 0, 0))
    wspech = pl.BlockSpec((1, 1, H), lambda i: (0, 0, 0))
    return pl.pallas_call(
        _logits_body,
        grid=(Tc // TB,),
        in_specs=[pl.BlockSpec((TB, Bb, D8), lambda i: (i, 0, 0)),
                  hspec, hspec, hspec, hspec,
                  wspec1, wspech, wspech, wspec1, wspech, wspech],
        out_specs=[pl.BlockSpec((TB, Bb), lambda i: (i, 0)),
                   pl.BlockSpec((TB, Bb), lambda i: (i, 0))],
        out_shape=[jax.ShapeDtypeStruct((Tc, Bb), F32),
                   jax.ShapeDtypeStruct((Tc, Bb), F32)],
    )(att, *mod_parts, *mod2_parts,
      va('Watt1'), vh('Wmod1', 0), vh('Wmod1', 1),
      va('Watt2'), vh('Wmod2', 0), vh('Wmod2', 1))


def _lsm_body(l1_ref, l2_ref, m_ref, o1_ref, o2_ref):
    m = m_ref[...] > 0
    for lr, orr in ((l1_ref, o1_ref), (l2_ref, o2_ref)):
        x = jnp.where(m, lr[...], NEGL)
        mx = jnp.max(x, axis=0, keepdims=True)
        e = jnp.exp(x - mx)
        s = jnp.sum(e, axis=0, keepdims=True)
        orr[...] = x - mx - jnp.log(s)


def _logsoftmax(l1, l2, cm_t):
    Tc, Bb = l1.shape
    return pl.pallas_call(
        _lsm_body,
        out_shape=[jax.ShapeDtypeStruct((Tc, Bb), F32),
                   jax.ShapeDtypeStruct((Tc, Bb), F32)],
    )(l1, l2, cm_t)


def kernel(cw_idxs, qw_idxs, bert_embeddings, max_context_len,
           max_question_len, device, params, word_vectors):
    p = params
    Bb, mc = cw_idxs.shape
    mq = qw_idxs.shape[1]
    cw = cw_idxs.astype(jnp.int32)
    qw = qw_idxs.astype(jnp.int32)
    c_mask = ((cw != 0) & (jnp.arange(mc) < max_context_len)[None, :]).astype(F32)
    q_mask = ((qw != 0) & (jnp.arange(mq) < max_question_len)[None, :]).astype(F32)
    cm_t = c_mask.T
    qm_t = q_mask.T

    idx_t = jnp.concatenate([cw, qw], axis=1).T.reshape(-1)
    ntok = idx_t.shape[0]
    npad = ((ntok + 255) // 256) * 256
    idx_pad = jnp.zeros((npad,), jnp.int32).at[:ntok].set(idx_t)

    tp = _project_table(word_vectors, p['Wproj'].T)
    e_all = _sc_gather(tp, idx_pad)[:ntok]

    bert_t = jnp.transpose(bert_embeddings, (1, 0, 2)).reshape(ntok, -1)
    x_all = _embed_hw(e_all, bert_t, p).reshape(mc + mq, Bb, H)
    c_emb = x_all[:mc]
    q_emb = x_all[mc:]

    cm_tb = cm_t.astype(BF16)
    qm_tb = qm_t.astype(BF16)
    cf, cb = _bilstm([c_emb], cm_tb, p['enc'], TB=80)
    qf, qb = _bilstm([q_emb], qm_tb, p['enc'], TB=mq)

    att = _attention([cf, cb], [qf, qb], c_mask, q_mask, p)

    m1f, m1b = _bilstm([att], cm_tb, p['mod1'], TB=80)
    m2f, m2b = _bilstm([m1f, m1b], cm_tb, p['mod2'], TB=80)
    mof, mob = _bilstm([m2f, m2b], cm_tb, p['out_rnn'], TB=80)

    l1, l2 = _logits(att, (m2f, m2b), (mof, mob), p)
    lp1, lp2 = _logsoftmax(l1, l2, cm_t)
    return lp1.T, lp2.T


# speedup vs baseline: 1.0197x; 1.0197x over previous
import functools

import jax
import jax.numpy as jnp
from jax import lax
from jax.experimental import pallas as pl
from jax.experimental.pallas import tpu as pltpu
from jax.experimental.pallas import tpu_sc as plsc

F32 = jnp.float32
BF16 = jnp.bfloat16
NEGL = -1e30
H = 128
H4 = 512


def _mm(a, b):
    return jnp.dot(a.astype(BF16), b.astype(BF16), preferred_element_type=F32)


def _tableproj_body(wv_ref, wt_ref, out_ref):
    out_ref[...] = _mm(wv_ref[...], wt_ref[...])


def _project_table(wv, wprojT):
    Vv, Dd = wv.shape
    Hh = wprojT.shape[1]
    blk = 10000
    return pl.pallas_call(
        _tableproj_body,
        grid=(Vv // blk,),
        in_specs=[pl.BlockSpec((blk, Dd), lambda i: (i, 0)),
                  pl.BlockSpec((Dd, Hh), lambda i: (0, 0))],
        out_specs=pl.BlockSpec((blk, Hh), lambda i: (i, 0)),
        out_shape=jax.ShapeDtypeStruct((Vv, Hh), F32),
    )(wv, wprojT)


def _sc_gather(table, idx):
    info = plsc.get_sparse_core_info()
    nc, ns = info.num_cores, info.num_subcores
    nw = nc * ns
    n = idx.shape[0]
    bpw = n // nw
    Dd = table.shape[1]
    dt = table.dtype
    row_b = Dd * table.dtype.itemsize
    buf_rows = min(bpw, (400 * 1024 // row_b) // 8 * 8)
    chunks = []
    off = 0
    while off < bpw:
        sz = min(buf_rows, bpw - off)
        chunks.append((off, sz))
        off += sz
    mesh = plsc.VectorSubcoreMesh(core_axis_name="c", subcore_axis_name="s")

    @functools.partial(
        pl.kernel, mesh=mesh,
        out_type=jax.ShapeDtypeStruct((n, Dd), dt),
        scratch_types=[pltpu.VMEM((bpw,), jnp.int32),
                       pltpu.VMEM((buf_rows, Dd), dt),
                       pltpu.SemaphoreType.DMA],
    )
    def gk(table_hbm, idx_hbm, out_hbm, idx_v, rows_v, sem):
        wid = lax.axis_index("s") * nc + lax.axis_index("c")
        base = wid * bpw
        pltpu.sync_copy(idx_hbm.at[pl.ds(base, bpw)], idx_v)
        for off, sz in chunks:
            pltpu.async_copy(table_hbm.at[idx_v.at[pl.ds(off, sz)]],
                             rows_v.at[pl.ds(0, sz)], sem).wait()
            pltpu.sync_copy(rows_v.at[pl.ds(0, sz)],
                            out_hbm.at[pl.ds(base + off, sz)])

    return gk(table, idx)


def _embed_body(e_ref, bert_ref, wg1, bg1, wt1, bt1, wg2, bg2, wt2, bt2,
                wb, bb, out_ref):
    x = e_ref[...]
    for wg, bg, wt, bt in ((wg1, bg1, wt1, bt1), (wg2, bg2, wt2, bt2)):
        g = jax.nn.sigmoid(_mm(x, wg[...]) + bg[...])
        t = jnp.maximum(_mm(x, wt[...]) + bt[...], 0.0)
        x = g * t + (1.0 - g) * x
    bh = jnp.maximum(_mm(bert_ref[...], wb[...]) + bb[...], 0.0)
    out_ref[...] = (x * (1.0 + bh)).astype(BF16)


def _embed_hw(e_all, bert_all, p):
    n, Hh = e_all.shape
    Db = bert_all.shape[1]
    blk = 1440
    w = lambda k: p[k].T
    b = lambda k: p[k].reshape(1, -1)
    args = (w('Wg1'), b('bg1'), w('Wt1'), b('bt1'),
            w('Wg2'), b('bg2'), w('Wt2'), b('bt2'),
            w('Wbert'), b('bbert'))
    return pl.pallas_call(
        _embed_body,
        grid=(n // blk,),
        in_specs=[pl.BlockSpec((blk, Hh), lambda i: (i, 0)),
                  pl.BlockSpec((blk, Db), lambda i: (i, 0))]
                 + [pl.BlockSpec(a.shape, lambda i: (0, 0)) for a in args],
        out_specs=pl.BlockSpec((blk, Hh), lambda i: (i, 0)),
        out_shape=jax.ShapeDtypeStruct((n, Hh), BF16),
    )(e_all, bert_all, *args)


def _bilstm_body(TB, Bb, dins, *refs):
    np_ = len(dins)
    xf_refs = refs[0:np_]
    xb_refs = refs[np_:2 * np_]
    mf_ref, mb_ref = refs[2 * np_:2 * np_ + 2]
    wf_refs = refs[2 * np_ + 2:3 * np_ + 2]
    wb_refs = refs[3 * np_ + 2:4 * np_ + 2]
    bf_ref, bb_ref, wc_ref = refs[4 * np_ + 2:4 * np_ + 5]
    outf_ref, outb_ref, xpf_s, xpb_s, hc_s, wcb_s = refs[4 * np_ + 5:]
    j = pl.program_id(0)

    @pl.when(j == 0)
    def _():
        hc_s[...] = jnp.zeros_like(hc_s)

    wcb_s[...] = wc_ref[...].astype(BF16)

    xpf = bf_ref[...]
    xpb = bb_ref[...]
    for xr, wr, d in zip(xf_refs, wf_refs, dins):
        xpf = xpf + _mm(xr[...].reshape(TB * Bb, d), wr[...])
    for xr, wr, d in zip(xb_refs, wb_refs, dins):
        xpb = xpb + _mm(xr[...].reshape(TB * Bb, d), wr[...])
    xpf_s[...] = xpf.reshape(TB, Bb, H4)
    xpb_s[...] = xpb.reshape(TB, Bb, H4)

    def step(k, _):
        kk = TB - 1 - k
        h = hc_s[0]
        c = hc_s[1]
        z64 = jnp.dot(h.astype(BF16), wcb_s[...], preferred_element_type=F32)
        zf = z64[0:Bb, 0:H4] + xpf_s[k]
        zb = z64[Bb:2 * Bb, H4:2 * H4] + xpb_s[kk]
        z = jnp.concatenate([zf, zb], axis=0)
        i_ = jax.nn.sigmoid(z[:, 0:H])
        f_ = jax.nn.sigmoid(z[:, H:2 * H])
        g_ = jnp.tanh(z[:, 2 * H:3 * H])
        o_ = jax.nn.sigmoid(z[:, 3 * H:4 * H])
        c_new = f_ * c + i_ * g_
        h_new = o_ * jnp.tanh(c_new)
        hc_s[0] = h_new
        hc_s[1] = c_new
        outf_ref[k] = h_new[0:Bb].astype(BF16)
        outb_ref[kk] = h_new[Bb:2 * Bb].astype(BF16)
        return 0

    lax.fori_loop(0, TB, step, 0, unroll=16)
    outf_ref[...] = outf_ref[...] * mf_ref[...][:, :, None]
    outb_ref[...] = outb_ref[...] * mb_ref[...][:, :, None]


def _bilstm(x_parts, mask_t, p, TB):
    T, Bb = x_parts[0].shape[:2]
    dins = tuple(x.shape[2] for x in x_parts)
    G = T // TB
    wfT = p['Wih_f'].T
    wbT = p['Wih_b'].T
    offs = [0]
    for d in dins:
        offs.append(offs[-1] + d)
    wf_parts = [wfT[offs[i]:offs[i + 1]] for i in range(len(dins))]
    wb_parts = [wbT[offs[i]:offs[i + 1]] for i in range(len(dins))]
    bf = p['b_f'].reshape(1, -1)
    bb = p['b_b'].reshape(1, -1)
    wc = jnp.concatenate([p['Whh_f'].T, p['Whh_b'].T], axis=1)
    body = functools.partial(_bilstm_body, TB, Bb, dins)
    xspec_f = [pl.BlockSpec((TB, Bb, d), lambda j: (j, 0, 0)) for d in dins]
    xspec_b = [pl.BlockSpec((TB, Bb, d), lambda j, G=G: (G - 1 - j, 0, 0))
               for d in dins]
    wspec = [pl.BlockSpec((d, H4), lambda j: (0, 0)) for d in dins]
    outf, outb = pl.pallas_call(
        body,
        grid=(G,),
        in_specs=xspec_f + xspec_b + [
            pl.BlockSpec((TB, Bb), lambda j: (j, 0)),
            pl.BlockSpec((TB, Bb), lambda j, G=G: (G - 1 - j, 0)),
        ] + wspec + wspec + [
            pl.BlockSpec((1, H4), lambda j: (0, 0)),
            pl.BlockSpec((1, H4), lambda j: (0, 0)),
            pl.BlockSpec((H, 2 * H4), lambda j: (0, 0)),
        ],
        out_specs=[
            pl.BlockSpec((TB, Bb, H), lambda j: (j, 0, 0)),
            pl.BlockSpec((TB, Bb, H), lambda j, G=G: (G - 1 - j, 0, 0)),
        ],
        out_shape=[jax.ShapeDtypeStruct((T, Bb, H), BF16),
                   jax.ShapeDtypeStruct((T, Bb, H), BF16)],
        scratch_shapes=[pltpu.VMEM((TB, Bb, H4), F32),
                        pltpu.VMEM((TB, Bb, H4), F32),
                        pltpu.VMEM((2, 2 * Bb, H), F32),
                        pltpu.VMEM((H, 2 * H4), BF16)],
    )(*x_parts, *x_parts, *(mask_t, mask_t), *wf_parts, *wb_parts, bf, bb, wc)
    return outf, outb


def _att_body(GB, Tc, Tq, cf_ref, cb_ref, qf_ref, qb_ref, cm_ref, qm_ref,
              wq_ref, bq_ref, out_ref):
    cv = jnp.concatenate([cf_ref[...], cb_ref[...]], axis=2)
    c = jnp.transpose(cv, (1, 0, 2))
    q = jnp.transpose(jnp.concatenate([qf_ref[...], qb_ref[...]], axis=2),
                      (1, 0, 2))
    cm = cm_ref[0]
    qm = qm_ref[0]
    D2 = c.shape[2]
    qp = jnp.tanh(_mm(q.reshape(GB * Tq, D2), wq_ref[...]).reshape(GB, Tq, D2)
                  + bq_ref[...])
    Lg = lax.dot_general(c, qp.astype(BF16),
                         (((2,), (2,)), ((0,), (0,))),
                         preferred_element_type=F32)
    La = jnp.where(qm[:, None, :] > 0, Lg, NEGL)
    A = jax.nn.softmax(La, axis=2)
    Lb = jnp.where(cm[:, :, None] > 0, Lg, NEGL)
    Bm = jax.nn.softmax(Lb, axis=1)
    c2q = lax.dot_general(A.astype(BF16), qp.astype(BF16),
                          (((2,), (1,)), ((0,), (0,))),
                          preferred_element_type=F32)
    q2c = lax.dot_general(Bm.astype(BF16), c,
                          (((1,), (1,)), ((0,), (0,))),
                          preferred_element_type=F32)
    coatt = lax.dot_general(A.astype(BF16), q2c.astype(BF16),
                            (((2,), (1,)), ((0,), (0,))),
                            preferred_element_type=F32)
    c2q_t = jnp.transpose(c2q, (1, 0, 2))
    coatt_t = jnp.transpose(coatt, (1, 0, 2))
    cv32 = cv.astype(F32)
    out_ref[:, :, 0:D2] = cv
    out_ref[:, :, D2:2 * D2] = c2q_t.astype(BF16)
    out_ref[:, :, 2 * D2:3 * D2] = (cv32 * c2q_t).astype(BF16)
    out_ref[:, :, 3 * D2:4 * D2] = (cv32 * coatt_t).astype(BF16)


def _attention(c_parts, q_parts, cm_b, qm_b, p):
    Tc, Bb, Hh = c_parts[0].shape
    Tq = q_parts[0].shape[0]
    D2 = 2 * Hh
    GB = 16
    wq = p['Wq'].T
    bq = p['bq'].reshape(1, 1, -1)
    cm3 = cm_b.reshape(Bb // GB, GB, Tc)
    qm3 = qm_b.reshape(Bb // GB, GB, Tq)
    body = functools.partial(_att_body, GB, Tc, Tq)
    return pl.pallas_call(
        body,
        grid=(Bb // GB,),
        in_specs=[
            pl.BlockSpec((Tc, GB, Hh), lambda i: (0, i, 0)),
            pl.BlockSpec((Tc, GB, Hh), lambda i: (0, i, 0)),
            pl.BlockSpec((Tq, GB, Hh), lambda i: (0, i, 0)),
            pl.BlockSpec((Tq, GB, Hh), lambda i: (0, i, 0)),
            pl.BlockSpec((1, GB, Tc), lambda i: (i, 0, 0)),
            pl.BlockSpec((1, GB, Tq), lambda i: (i, 0, 0)),
            pl.BlockSpec((D2, D2), lambda i: (0, 0)),
            pl.BlockSpec((1, 1, D2), lambda i: (0, 0, 0)),
        ],
        out_specs=pl.BlockSpec((Tc, GB, 4 * D2), lambda i: (0, i, 0)),
        out_shape=jax.ShapeDtypeStruct((Tc, Bb, 4 * D2), BF16),
    )(*c_parts, *q_parts, cm3, qm3, wq, bq)


def _logits_body(att_ref, m2f_ref, m2b_ref, mof_ref, mob_ref,
                 wa1, wm1a, wm1b, wa2, wm2a, wm2b, l1_ref, l2_ref):
    att = att_ref[...]
    l1_ref[...] = (jnp.sum(att * wa1[...], axis=2)
                   + jnp.sum(m2f_ref[...] * wm1a[...], axis=2)
                   + jnp.sum(m2b_ref[...] * wm1b[...], axis=2))
    l2_ref[...] = (jnp.sum(att * wa2[...], axis=2)
                   + jnp.sum(mof_ref[...] * wm2a[...], axis=2)
                   + jnp.sum(mob_ref[...] * wm2b[...], axis=2))


def _logits(att, mod_parts, mod2_parts, p):
    Tc, Bb, D8 = att.shape
    TB = 80
    va = lambda k: p[k].reshape(1, 1, -1)
    vh = lambda k, s: p[k].reshape(-1)[s * H:(s + 1) * H].reshape(1, 1, H)
    hspec = pl.BlockSpec((TB, Bb, H), lambda i: (i, 0, 0))
    wspec1 = pl.BlockSpec((1, 1, D8), lambda i: (0, 0, 0))
    wspech = pl.BlockSpec((1, 1, H), lambda i: (0, 0, 0))
    return pl.pallas_call(
        _logits_body,
        grid=(Tc // TB,),
        in_specs=[pl.BlockSpec((TB, Bb, D8), lambda i: (i, 0, 0)),
                  hspec, hspec, hspec, hspec,
                  wspec1, wspech, wspech, wspec1, wspech, wspech],
        out_specs=[pl.BlockSpec((TB, Bb), lambda i: (i, 0)),
                   pl.BlockSpec((TB, Bb), lambda i: (i, 0))],
        out_shape=[jax.ShapeDtypeStruct((Tc, Bb), F32),
                   jax.ShapeDtypeStruct((Tc, Bb), F32)],
    )(att, *mod_parts, *mod2_parts,
      va('Watt1'), vh('Wmod1', 0), vh('Wmod1', 1),
      va('Watt2'), vh('Wmod2', 0), vh('Wmod2', 1))


def _lsm_body(l1_ref, l2_ref, m_ref, o1_ref, o2_ref):
    m = m_ref[...] > 0
    for lr, orr in ((l1_ref, o1_ref), (l2_ref, o2_ref)):
        x = jnp.where(m, lr[...], NEGL)
        mx = jnp.max(x, axis=0, keepdims=True)
        e = jnp.exp(x - mx)
        s = jnp.sum(e, axis=0, keepdims=True)
        orr[...] = x - mx - jnp.log(s)


def _logsoftmax(l1, l2, cm_t):
    Tc, Bb = l1.shape
    return pl.pallas_call(
        _lsm_body,
        out_shape=[jax.ShapeDtypeStruct((Tc, Bb), F32),
                   jax.ShapeDtypeStruct((Tc, Bb), F32)],
    )(l1, l2, cm_t)


def kernel(cw_idxs, qw_idxs, bert_embeddings, max_context_len,
           max_question_len, device, params, word_vectors):
    p = params
    Bb, mc = cw_idxs.shape
    mq = qw_idxs.shape[1]
    cw = cw_idxs.astype(jnp.int32)
    qw = qw_idxs.astype(jnp.int32)
    c_mask = ((cw != 0) & (jnp.arange(mc) < max_context_len)[None, :]).astype(F32)
    q_mask = ((qw != 0) & (jnp.arange(mq) < max_question_len)[None, :]).astype(F32)
    cm_t = c_mask.T
    qm_t = q_mask.T

    idx_t = jnp.concatenate([cw, qw], axis=1).T.reshape(-1)
    ntok = idx_t.shape[0]
    npad = ((ntok + 255) // 256) * 256
    idx_pad = jnp.zeros((npad,), jnp.int32).at[:ntok].set(idx_t)

    tp = _project_table(word_vectors, p['Wproj'].T)
    e_all = _sc_gather(tp, idx_pad)[:ntok]

    bert_t = jnp.transpose(bert_embeddings, (1, 0, 2)).reshape(ntok, -1)
    x_all = _embed_hw(e_all, bert_t, p).reshape(mc + mq, Bb, H)
    c_emb = x_all[:mc]
    q_emb = x_all[mc:]

    cm_tb = cm_t.astype(BF16)
    qm_tb = qm_t.astype(BF16)
    cf, cb = _bilstm([c_emb], cm_tb, p['enc'], TB=80)
    qf, qb = _bilstm([q_emb], qm_tb, p['enc'], TB=mq)

    att = _attention([cf, cb], [qf, qb], c_mask, q_mask, p)

    m1f, m1b = _bilstm([att], cm_tb, p['mod1'], TB=80)
    m2f, m2b = _bilstm([m1f, m1b], cm_tb, p['mod2'], TB=80)
    mof, mob = _bilstm([m2f, m2b], cm_tb, p['out_rnn'], TB=80)

    l1, l2 = _logits(att, (m2f, m2b), (mof, mob), p)
    lp1, lp2 = _logsoftmax(l1, l2, cm_t)
    return lp1.T, lp2.T


# speedup vs baseline: 1.0200x; 1.0003x over previous
import functools

import jax
import jax.numpy as jnp
from jax import lax
from jax.experimental import pallas as pl
from jax.experimental.pallas import tpu as pltpu
from jax.experimental.pallas import tpu_sc as plsc

F32 = jnp.float32
BF16 = jnp.bfloat16
NEGL = -1e30
H = 128
H4 = 512


def _mm(a, b):
    return jnp.dot(a.astype(BF16), b.astype(BF16), preferred_element_type=F32)


def _tableproj_body(wv_ref, wt_ref, out_ref):
    out_ref[...] = _mm(wv_ref[...], wt_ref[...])


def _project_table(wv, wprojT):
    Vv, Dd = wv.shape
    Hh = wprojT.shape[1]
    blk = 10000
    return pl.pallas_call(
        _tableproj_body,
        grid=(Vv // blk,),
        in_specs=[pl.BlockSpec((blk, Dd), lambda i: (i, 0)),
                  pl.BlockSpec((Dd, Hh), lambda i: (0, 0))],
        out_specs=pl.BlockSpec((blk, Hh), lambda i: (i, 0)),
        out_shape=jax.ShapeDtypeStruct((Vv, Hh), F32),
    )(wv, wprojT)


def _sc_gather(table, idx):
    info = plsc.get_sparse_core_info()
    nc, ns = info.num_cores, info.num_subcores
    nw = nc * ns
    n = idx.shape[0]
    bpw = n // nw
    Dd = table.shape[1]
    dt = table.dtype
    row_b = Dd * table.dtype.itemsize
    buf_rows = min(bpw, (400 * 1024 // row_b) // 8 * 8)
    chunks = []
    off = 0
    while off < bpw:
        sz = min(buf_rows, bpw - off)
        chunks.append((off, sz))
        off += sz
    mesh = plsc.VectorSubcoreMesh(core_axis_name="c", subcore_axis_name="s")

    @functools.partial(
        pl.kernel, mesh=mesh,
        out_type=jax.ShapeDtypeStruct((n, Dd), dt),
        scratch_types=[pltpu.VMEM((bpw,), jnp.int32),
                       pltpu.VMEM((buf_rows, Dd), dt),
                       pltpu.SemaphoreType.DMA],
    )
    def gk(table_hbm, idx_hbm, out_hbm, idx_v, rows_v, sem):
        wid = lax.axis_index("s") * nc + lax.axis_index("c")
        base = wid * bpw
        pltpu.sync_copy(idx_hbm.at[pl.ds(base, bpw)], idx_v)
        for off, sz in chunks:
            pltpu.async_copy(table_hbm.at[idx_v.at[pl.ds(off, sz)]],
                             rows_v.at[pl.ds(0, sz)], sem).wait()
            pltpu.sync_copy(rows_v.at[pl.ds(0, sz)],
                            out_hbm.at[pl.ds(base + off, sz)])

    return gk(table, idx)


def _embed_body(e_ref, bert_ref, wg1, bg1, wt1, bt1, wg2, bg2, wt2, bt2,
                wb, bb, out_ref):
    x = e_ref[...]
    for wg, bg, wt, bt in ((wg1, bg1, wt1, bt1), (wg2, bg2, wt2, bt2)):
        g = jax.nn.sigmoid(_mm(x, wg[...]) + bg[...])
        t = jnp.maximum(_mm(x, wt[...]) + bt[...], 0.0)
        x = g * t + (1.0 - g) * x
    bh = jnp.maximum(_mm(bert_ref[...], wb[...]) + bb[...], 0.0)
    out_ref[...] = (x * (1.0 + bh)).astype(BF16)


def _embed_hw(e_all, bert_all, p):
    n, Hh = e_all.shape
    Db = bert_all.shape[1]
    blk = 1440
    w = lambda k: p[k].T
    b = lambda k: p[k].reshape(1, -1)
    args = (w('Wg1'), b('bg1'), w('Wt1'), b('bt1'),
            w('Wg2'), b('bg2'), w('Wt2'), b('bt2'),
            w('Wbert'), b('bbert'))
    return pl.pallas_call(
        _embed_body,
        grid=(n // blk,),
        in_specs=[pl.BlockSpec((blk, Hh), lambda i: (i, 0)),
                  pl.BlockSpec((blk, Db), lambda i: (i, 0))]
                 + [pl.BlockSpec(a.shape, lambda i: (0, 0)) for a in args],
        out_specs=pl.BlockSpec((blk, Hh), lambda i: (i, 0)),
        out_shape=jax.ShapeDtypeStruct((n, Hh), BF16),
    )(e_all, bert_all, *args)


def _bilstm_body(TB, Bb, dins, has_lw, *refs):
    np_ = len(dins)
    xf_refs = refs[0:np_]
    xb_refs = refs[np_:2 * np_]
    mf_ref, mb_ref = refs[2 * np_:2 * np_ + 2]
    wf_refs = refs[2 * np_ + 2:3 * np_ + 2]
    wb_refs = refs[3 * np_ + 2:4 * np_ + 2]
    bf_ref, bb_ref, wc_ref = refs[4 * np_ + 2:4 * np_ + 5]
    k0 = 4 * np_ + 5
    if has_lw:
        wla_ref, wlb_ref = refs[k0:k0 + 2]
        (outf_ref, outb_ref, lf_ref, lb_ref,
         xpf_s, xpb_s, hc_s, wcb_s) = refs[k0 + 2:]
    else:
        outf_ref, outb_ref, xpf_s, xpb_s, hc_s, wcb_s = refs[k0:]
    j = pl.program_id(0)

    @pl.when(j == 0)
    def _():
        hc_s[...] = jnp.zeros_like(hc_s)

    wcb_s[...] = wc_ref[...].astype(BF16)

    xpf = bf_ref[...]
    xpb = bb_ref[...]
    for xr, wr, d in zip(xf_refs, wf_refs, dins):
        xpf = xpf + _mm(xr[...].reshape(TB * Bb, d), wr[...])
    for xr, wr, d in zip(xb_refs, wb_refs, dins):
        xpb = xpb + _mm(xr[...].reshape(TB * Bb, d), wr[...])
    xpf_s[...] = xpf.reshape(TB, Bb, H4)
    xpb_s[...] = xpb.reshape(TB, Bb, H4)

    def step(k, _):
        kk = TB - 1 - k
        h = hc_s[0]
        c = hc_s[1]
        z64 = jnp.dot(h.astype(BF16), wcb_s[...], preferred_element_type=F32)
        zf = z64[0:Bb, 0:H4] + xpf_s[k]
        zb = z64[Bb:2 * Bb, H4:2 * H4] + xpb_s[kk]
        z = jnp.concatenate([zf, zb], axis=0)
        i_ = jax.nn.sigmoid(z[:, 0:H])
        f_ = jax.nn.sigmoid(z[:, H:2 * H])
        g_ = jnp.tanh(z[:, 2 * H:3 * H])
        o_ = jax.nn.sigmoid(z[:, 3 * H:4 * H])
        c_new = f_ * c + i_ * g_
        h_new = o_ * jnp.tanh(c_new)
        hc_s[0] = h_new
        hc_s[1] = c_new
        outf_ref[k] = h_new[0:Bb].astype(BF16)
        outb_ref[kk] = h_new[Bb:2 * Bb].astype(BF16)
        return 0

    lax.fori_loop(0, TB, step, 0, unroll=16)
    outf_ref[...] = outf_ref[...] * mf_ref[...][:, :, None]
    outb_ref[...] = outb_ref[...] * mb_ref[...][:, :, None]
    if has_lw:
        lf_ref[...] = jnp.sum(outf_ref[...] * wla_ref[...], axis=2)
        lb_ref[...] = jnp.sum(outb_ref[...] * wlb_ref[...], axis=2)


def _bilstm(x_parts, mask_t, p, TB, logit_w=None):
    T, Bb = x_parts[0].shape[:2]
    dins = tuple(x.shape[2] for x in x_parts)
    G = T // TB
    wfT = p['Wih_f'].T
    wbT = p['Wih_b'].T
    offs = [0]
    for d in dins:
        offs.append(offs[-1] + d)
    wf_parts = [wfT[offs[i]:offs[i + 1]] for i in range(len(dins))]
    wb_parts = [wbT[offs[i]:offs[i + 1]] for i in range(len(dins))]
    bf = p['b_f'].reshape(1, -1)
    bb = p['b_b'].reshape(1, -1)
    wc = jnp.concatenate([p['Whh_f'].T, p['Whh_b'].T], axis=1)
    body = functools.partial(_bilstm_body, TB, Bb, dins, logit_w is not None)
    xspec_f = [pl.BlockSpec((TB, Bb, d), lambda j: (j, 0, 0)) for d in dins]
    xspec_b = [pl.BlockSpec((TB, Bb, d), lambda j, G=G: (G - 1 - j, 0, 0))
               for d in dins]
    wspec = [pl.BlockSpec((d, H4), lambda j: (0, 0)) for d in dins]
    in_specs = xspec_f + xspec_b + [
        pl.BlockSpec((TB, Bb), lambda j: (j, 0)),
        pl.BlockSpec((TB, Bb), lambda j, G=G: (G - 1 - j, 0)),
    ] + wspec + wspec + [
        pl.BlockSpec((1, H4), lambda j: (0, 0)),
        pl.BlockSpec((1, H4), lambda j: (0, 0)),
        pl.BlockSpec((H, 2 * H4), lambda j: (0, 0)),
    ]
    out_specs = [
        pl.BlockSpec((TB, Bb, H), lambda j: (j, 0, 0)),
        pl.BlockSpec((TB, Bb, H), lambda j, G=G: (G - 1 - j, 0, 0)),
    ]
    out_shape = [jax.ShapeDtypeStruct((T, Bb, H), BF16),
                 jax.ShapeDtypeStruct((T, Bb, H), BF16)]
    extra = ()
    if logit_w is not None:
        in_specs += [pl.BlockSpec((1, 1, H), lambda j: (0, 0, 0))] * 2
        out_specs += [pl.BlockSpec((TB, Bb), lambda j: (j, 0)),
                      pl.BlockSpec((TB, Bb), lambda j, G=G: (G - 1 - j, 0))]
        out_shape += [jax.ShapeDtypeStruct((T, Bb), F32),
                      jax.ShapeDtypeStruct((T, Bb), F32)]
        extra = tuple(logit_w)
    return pl.pallas_call(
        body,
        grid=(G,),
        in_specs=in_specs,
        out_specs=out_specs,
        out_shape=out_shape,
        scratch_shapes=[pltpu.VMEM((TB, Bb, H4), F32),
                        pltpu.VMEM((TB, Bb, H4), F32),
                        pltpu.VMEM((2, 2 * Bb, H), F32),
                        pltpu.VMEM((H, 2 * H4), BF16)],
    )(*x_parts, *x_parts, *(mask_t, mask_t), *wf_parts, *wb_parts,
      bf, bb, wc, *extra)


def _att_body(GB, Tc, Tq, cf_ref, cb_ref, qf_ref, qb_ref, cm_ref, qm_ref,
              wq_ref, bq_ref, wa1_ref, wa2_ref, out_ref, la1_ref, la2_ref):
    cv = jnp.concatenate([cf_ref[...], cb_ref[...]], axis=2)
    c = jnp.transpose(cv, (1, 0, 2))
    q = jnp.transpose(jnp.concatenate([qf_ref[...], qb_ref[...]], axis=2),
                      (1, 0, 2))
    cm = cm_ref[0]
    qm = qm_ref[0]
    D2 = c.shape[2]
    qp = jnp.tanh(_mm(q.reshape(GB * Tq, D2), wq_ref[...]).reshape(GB, Tq, D2)
                  + bq_ref[...])
    Lg = lax.dot_general(c, qp.astype(BF16),
                         (((2,), (2,)), ((0,), (0,))),
                         preferred_element_type=F32)
    La = jnp.where(qm[:, None, :] > 0, Lg, NEGL)
    A = jax.nn.softmax(La, axis=2)
    Lb = jnp.where(cm[:, :, None] > 0, Lg, NEGL)
    Bm = jax.nn.softmax(Lb, axis=1)
    c2q = lax.dot_general(A.astype(BF16), qp.astype(BF16),
                          (((2,), (1,)), ((0,), (0,))),
                          preferred_element_type=F32)
    q2c = lax.dot_general(Bm.astype(BF16), c,
                          (((1,), (1,)), ((0,), (0,))),
                          preferred_element_type=F32)
    coatt = lax.dot_general(A.astype(BF16), q2c.astype(BF16),
                            (((2,), (1,)), ((0,), (0,))),
                            preferred_element_type=F32)
    c2q_t = jnp.transpose(c2q, (1, 0, 2))
    coatt_t = jnp.transpose(coatt, (1, 0, 2))
    cv32 = cv.astype(F32)
    p3 = cv32 * c2q_t
    p4 = cv32 * coatt_t
    out_ref[:, :, 0:D2] = cv
    out_ref[:, :, D2:2 * D2] = c2q_t.astype(BF16)
    out_ref[:, :, 2 * D2:3 * D2] = p3.astype(BF16)
    out_ref[:, :, 3 * D2:4 * D2] = p4.astype(BF16)
    for wref, lref in ((wa1_ref, la1_ref), (wa2_ref, la2_ref)):
        wv_ = wref[...]
        la = (jnp.sum(cv32 * wv_[:, :, 0:D2], axis=2)
              + jnp.sum(c2q_t * wv_[:, :, D2:2 * D2], axis=2)
              + jnp.sum(p3 * wv_[:, :, 2 * D2:3 * D2], axis=2)
              + jnp.sum(p4 * wv_[:, :, 3 * D2:4 * D2], axis=2))
        lref[...] = la.reshape(1, Tc, GB)


def _attention(c_parts, q_parts, cm_b, qm_b, p):
    Tc, Bb, Hh = c_parts[0].shape
    Tq = q_parts[0].shape[0]
    D2 = 2 * Hh
    GB = 8
    wq = p['Wq'].T
    bq = p['bq'].reshape(1, 1, -1)
    cm3 = cm_b.reshape(Bb // GB, GB, Tc)
    qm3 = qm_b.reshape(Bb // GB, GB, Tq)
    wa1 = p['Watt1'].reshape(1, 1, -1)
    wa2 = p['Watt2'].reshape(1, 1, -1)
    ng = Bb // GB
    body = functools.partial(_att_body, GB, Tc, Tq)
    att, la1, la2 = pl.pallas_call(
        body,
        grid=(ng,),
        in_specs=[
            pl.BlockSpec((Tc, GB, Hh), lambda i: (0, i, 0)),
            pl.BlockSpec((Tc, GB, Hh), lambda i: (0, i, 0)),
            pl.BlockSpec((Tq, GB, Hh), lambda i: (0, i, 0)),
            pl.BlockSpec((Tq, GB, Hh), lambda i: (0, i, 0)),
            pl.BlockSpec((1, GB, Tc), lambda i: (i, 0, 0)),
            pl.BlockSpec((1, GB, Tq), lambda i: (i, 0, 0)),
            pl.BlockSpec((D2, D2), lambda i: (0, 0)),
            pl.BlockSpec((1, 1, D2), lambda i: (0, 0, 0)),
            pl.BlockSpec((1, 1, 4 * D2), lambda i: (0, 0, 0)),
            pl.BlockSpec((1, 1, 4 * D2), lambda i: (0, 0, 0)),
        ],
        out_specs=[pl.BlockSpec((Tc, GB, 4 * D2), lambda i: (0, i, 0)),
                   pl.BlockSpec((1, Tc, GB), lambda i: (i, 0, 0)),
                   pl.BlockSpec((1, Tc, GB), lambda i: (i, 0, 0))],
        out_shape=[jax.ShapeDtypeStruct((Tc, Bb, 4 * D2), BF16),
                   jax.ShapeDtypeStruct((ng, Tc, GB), F32),
                   jax.ShapeDtypeStruct((ng, Tc, GB), F32)],
    )(*c_parts, *q_parts, cm3, qm3, wq, bq, wa1, wa2)
    la1 = jnp.transpose(la1, (1, 0, 2)).reshape(Tc, Bb)
    la2 = jnp.transpose(la2, (1, 0, 2)).reshape(Tc, Bb)
    return att, la1, la2


def _lsm_body(la1, lf1, lb1, la2, lf2, lb2, m_ref, o1_ref, o2_ref):
    m = m_ref[...] > 0
    for pa, pf, pb, orr in ((la1, lf1, lb1, o1_ref), (la2, lf2, lb2, o2_ref)):
        x = jnp.where(m, pa[...] + pf[...] + pb[...], NEGL)
        mx = jnp.max(x, axis=0, keepdims=True)
        e = jnp.exp(x - mx)
        s = jnp.sum(e, axis=0, keepdims=True)
        orr[...] = x - mx - jnp.log(s)


def _logsoftmax(l1_parts, l2_parts, cm_t):
    Tc, Bb = cm_t.shape
    return pl.pallas_call(
        _lsm_body,
        out_shape=[jax.ShapeDtypeStruct((Tc, Bb), F32),
                   jax.ShapeDtypeStruct((Tc, Bb), F32)],
    )(*l1_parts, *l2_parts, cm_t)


def kernel(cw_idxs, qw_idxs, bert_embeddings, max_context_len,
           max_question_len, device, params, word_vectors):
    p = params
    Bb, mc = cw_idxs.shape
    mq = qw_idxs.shape[1]
    cw = cw_idxs.astype(jnp.int32)
    qw = qw_idxs.astype(jnp.int32)
    c_mask = ((cw != 0) & (jnp.arange(mc) < max_context_len)[None, :]).astype(F32)
    q_mask = ((qw != 0) & (jnp.arange(mq) < max_question_len)[None, :]).astype(F32)
    cm_t = c_mask.T
    qm_t = q_mask.T

    idx_t = jnp.concatenate([cw, qw], axis=1).T.reshape(-1)
    ntok = idx_t.shape[0]
    npad = ((ntok + 255) // 256) * 256
    idx_pad = jnp.zeros((npad,), jnp.int32).at[:ntok].set(idx_t)

    tp = _project_table(word_vectors, p['Wproj'].T)
    e_all = _sc_gather(tp, idx_pad)[:ntok]

    bert_t = jnp.transpose(bert_embeddings, (1, 0, 2)).reshape(ntok, -1)
    x_all = _embed_hw(e_all, bert_t, p).reshape(mc + mq, Bb, H)
    c_emb = x_all[:mc]
    q_emb = x_all[mc:]

    cm_tb = cm_t.astype(BF16)
    qm_tb = qm_t.astype(BF16)
    cf, cb = _bilstm([c_emb], cm_tb, p['enc'], TB=80)
    qf, qb = _bilstm([q_emb], qm_tb, p['enc'], TB=mq)

    att, la1, la2 = _attention([cf, cb], [qf, qb], c_mask, q_mask, p)

    wsplit = lambda k: (p[k].reshape(-1)[0:H].reshape(1, 1, H),
                        p[k].reshape(-1)[H:2 * H].reshape(1, 1, H))
    m1f, m1b = _bilstm([att], cm_tb, p['mod1'], TB=80)
    m2f, m2b, lf1, lb1 = _bilstm([m1f, m1b], cm_tb, p['mod2'], TB=80,
                                 logit_w=wsplit('Wmod1'))
    mof, mob, lf2, lb2 = _bilstm([m2f, m2b], cm_tb, p['out_rnn'], TB=80,
                                 logit_w=wsplit('Wmod2'))

    lp1, lp2 = _logsoftmax((la1, lf1, lb1), (la2, lf2, lb2), cm_t)
    return lp1.T, lp2.T
